# baseline (speedup 1.0000x reference)
import functools

import jax
import jax.numpy as jnp
from jax import lax
from jax.experimental import pallas as pl
from jax.experimental.pallas import tpu as pltpu
from jax.experimental.pallas import tpu_sc as plsc

NEG_FILL = -1000000000.0
L = 16
NGROUP = 8
NHALF = 5120
G = 128
NGRP = G // L

E_SC = 166400
SC_CH = 6400
TC_CH = 1600


def _build_idx(cbuf, idxbuf, b, goff):
    for q in range(NGRP):
        cv = cbuf[pl.ds(b * G + q * L, L)]
        idxbuf[pl.ds(q * L, L)] = cv + goff


def _process(rbuf, xg, aggr, b, row_lo):
    for q in range(NGRP):
        rv = rbuf[pl.ds(b * G + q * L, L)]
        rl = rv - row_lo
        ok = (rl >= 0) & (rl < NHALF)
        rl = jnp.where(ok, rl, NHALF)
        for lane in range(L):
            rr = rl[lane]
            e = q * L + lane
            aggr[rr, :] = jnp.maximum(aggr[rr, :], xg[e, :])


def _sc_body(row_hbm, col_hbm, xg_hbm, out_hbm,
             rbuf, cbuf, idx0, idx1, xg0, xg1, aggr, sem0, sem1):
    num_edges = row_hbm.shape[0]
    eh_edges = num_edges // 2
    n_nodes = xg_hbm.shape[0] // NGROUP
    ch = rbuf.shape[0]
    nch = eh_edges // ch
    nb = ch // G

    eh = lax.axis_index("c")
    s = lax.axis_index("s")
    g = s // 2
    nh = lax.rem(s, 2)
    goff = g * n_nodes
    row_lo = nh * NHALF
    ebase = eh * eh_edges

    neg = jnp.full((L,), NEG_FILL, jnp.float32)

    def init_body(i, carry):
        aggr[i, :] = neg
        return carry

    lax.fori_loop(0, NHALF + 1, init_body, 0)

    def start_gather(idxbuf, xg, sem):
        pltpu.async_copy(xg_hbm.at[idxbuf], xg, sem)

    def wait_gather(idxbuf, xg, sem):
        pltpu.make_async_copy(xg_hbm.at[idxbuf], xg, sem).wait()

    def chunk_body(t, carry):
        off = ebase + t * ch
        pltpu.sync_copy(row_hbm.at[pl.ds(off, ch)], rbuf)
        pltpu.sync_copy(col_hbm.at[pl.ds(off, ch)], cbuf)
        _build_idx(cbuf, idx0, 0, goff)
        start_gather(idx0, xg0, sem0)

        def pair_body(j, c2):
            b0 = j * 2
            _build_idx(cbuf, idx1, b0 + 1, goff)
            start_gather(idx1, xg1, sem1)
            wait_gather(idx0, xg0, sem0)
            _process(rbuf, xg0, aggr, b0, row_lo)
            _build_idx(cbuf, idx0, b0 + 2, goff)
            start_gather(idx0, xg0, sem0)
            wait_gather(idx1, xg1, sem1)
            _process(rbuf, xg1, aggr, b0 + 1, row_lo)
            return c2

        lax.fori_loop(0, nb // 2 - 1, pair_body, 0)
        _build_idx(cbuf, idx1, nb - 1, goff)
        start_gather(idx1, xg1, sem1)
        wait_gather(idx0, xg0, sem0)
        _process(rbuf, xg0, aggr, nb - 2, row_lo)
        wait_gather(idx1, xg1, sem1)
        _process(rbuf, xg1, aggr, nb - 1, row_lo)
        return carry

    lax.fori_loop(0, nch, chunk_body, 0)
    pltpu.sync_copy(aggr.at[pl.ds(0, NHALF)], out_hbm.at[eh, g, nh])


def _scatter_max_sc(row, col, xg):
    mesh = plsc.VectorSubcoreMesh(core_axis_name="c", subcore_axis_name="s")
    f = pl.kernel(
        _sc_body,
        mesh=mesh,
        compiler_params=pltpu.CompilerParams(use_tc_tiling_on_sc=False),
        out_type=jax.ShapeDtypeStruct((2, NGROUP, 2, NHALF, L), jnp.float32),
        scratch_types=[
            pltpu.VMEM((SC_CH,), jnp.int32),
            pltpu.VMEM((SC_CH,), jnp.int32),
            pltpu.VMEM((G,), jnp.int32),
            pltpu.VMEM((G,), jnp.int32),
            pltpu.VMEM((G, L), jnp.float32),
            pltpu.VMEM((G, L), jnp.float32),
            pltpu.VMEM((NHALF + 1, L), jnp.float32),
            pltpu.SemaphoreType.DMA,
            pltpu.SemaphoreType.DMA,
        ],
    )
    return f(row, col, xg)


def _tc_scatter_body(row_ref, col_ref, x_ref, aggr_ref, *, chunk):
    step = pl.program_id(0)

    @pl.when(step == 0)
    def _init():
        aggr_ref[...] = jnp.full_like(aggr_ref[...], NEG_FILL)

    def body(i, carry):
        r = row_ref[0, 0, i]
        c = col_ref[0, 0, i]
        xr = x_ref[c, :]
        aggr_ref[r, :] = jnp.maximum(aggr_ref[r, :], xr)
        return carry

    jax.lax.fori_loop(0, chunk, body, 0, unroll=4)


def _erf(z):
    s = jnp.sign(z)
    a = jnp.abs(z)
    t = 1.0 / (1.0 + 0.3275911 * a)
    poly = t * (0.254829592 + t * (-0.284496736 + t * (1.421413741
           + t * (-1.453152027 + t * 1.061405429))))
    return s * (1.0 - poly * jnp.exp(-a * a))


def _epilogue_body(p_ref, t_ref, x_ref, w_ref, b_ref, out_ref):
    a = jnp.maximum(jnp.maximum(p_ref[0], p_ref[1]), t_ref[...])
    a = jnp.where(a == NEG_FILL, 0.0, a) - x_ref[...]
    z = jnp.dot(a, w_ref[...], preferred_element_type=jnp.float32) + b_ref[...]
    out_ref[...] = 0.5 * z * (1.0 + _erf(z * 0.7071067811865476))


def kernel(x, edge_index, W, b):
    Bn, N, C = x.shape
    x_flat = x.reshape(N, C)
    xg = x_flat.reshape(N, NGROUP, L).transpose(1, 0, 2).reshape(NGROUP * N, L)

    E = edge_index.shape[1]
    e_tc = E - E_SC
    nb_tc = e_tc // TC_CH
    row_tc = lax.slice_in_dim(edge_index[0], E_SC, E).reshape(nb_tc, 1, TC_CH)
    col_tc = lax.slice_in_dim(edge_index[1], E_SC, E).reshape(nb_tc, 1, TC_CH)
    aggr_tc = pl.pallas_call(
        functools.partial(_tc_scatter_body, chunk=TC_CH),
        grid=(nb_tc,),
        in_specs=[
            pl.BlockSpec((1, 1, TC_CH), lambda i: (i, 0, 0),
                         memory_space=pltpu.SMEM),
            pl.BlockSpec((1, 1, TC_CH), lambda i: (i, 0, 0),
                         memory_space=pltpu.SMEM),
            pl.BlockSpec((N, C), lambda i: (0, 0)),
        ],
        out_specs=pl.BlockSpec((N, C), lambda i: (0, 0)),
        out_shape=jax.ShapeDtypeStruct((N, C), jnp.float32),
        compiler_params=pltpu.CompilerParams(
            dimension_semantics=("arbitrary",)),
    )(row_tc, col_tc, x_flat)

    row_sc = lax.slice_in_dim(edge_index[0], 0, E_SC)
    col_sc = lax.slice_in_dim(edge_index[1], 0, E_SC)
    out_sc = _scatter_max_sc(row_sc, col_sc, xg)

    partials = out_sc.transpose(0, 2, 3, 1, 4).reshape(2, 2 * NHALF, C)[:, :N]

    BN = 1000
    out = pl.pallas_call(
        _epilogue_body,
        grid=(N // BN,),
        in_specs=[
            pl.BlockSpec((2, BN, C), lambda i: (0, i, 0)),
            pl.BlockSpec((BN, C), lambda i: (i, 0)),
            pl.BlockSpec((BN, C), lambda i: (i, 0)),
            pl.BlockSpec((C, C), lambda i: (0, 0)),
            pl.BlockSpec((1, C), lambda i: (0, 0)),
        ],
        out_specs=pl.BlockSpec((BN, C), lambda i: (i, 0)),
        out_shape=jax.ShapeDtypeStruct((N, C), jnp.float32),
    )(partials, aggr_tc, x_flat, W, b.reshape(1, C))
    return out.reshape(Bn, N, C)


# speedup vs baseline: 1.0741x; 1.0741x over previous
import functools

import jax
import jax.numpy as jnp
from jax.experimental import pallas as pl
from jax.experimental.pallas import tpu as pltpu

NEG_FILL = -1000000000.0
K = 4


def _scatter_body(row_ref, col_ref, x_ref, aggr_ref, *, chunk):
    step = pl.program_id(0)

    @pl.when(step == 0)
    def _init():
        aggr_ref[...] = jnp.full_like(aggr_ref[...], NEG_FILL)

    def body(i, carry):
        for k in range(K):
            r = row_ref[0, 0, i * K + k]
            c = col_ref[0, 0, i * K + k]
            xr = x_ref[c, :]
            aggr_ref[k, r, :] = jnp.maximum(aggr_ref[k, r, :], xr)
        return carry

    jax.lax.fori_loop(0, chunk // K, body, 0, unroll=2)


def _erf(z):
    s = jnp.sign(z)
    a = jnp.abs(z)
    t = 1.0 / (1.0 + 0.3275911 * a)
    poly = t * (0.254829592 + t * (-0.284496736 + t * (1.421413741
           + t * (-1.453152027 + t * 1.061405429))))
    return s * (1.0 - poly * jnp.exp(-a * a))


def _epilogue_body(t_ref, x_ref, w_ref, b_ref, out_ref):
    a = t_ref[0]
    for k in range(1, K):
        a = jnp.maximum(a, t_ref[k])
    a = jnp.where(a == NEG_FILL, 0.0, a) - x_ref[...]
    z = jnp.dot(a, w_ref[...], preferred_element_type=jnp.float32) + b_ref[...]
    out_ref[...] = 0.5 * z * (1.0 + _erf(z * 0.7071067811865476))


def kernel(x, edge_index, W, b):
    Bn, N, C = x.shape
    x_flat = x.reshape(N, C)
    E = edge_index.shape[1]
    CHUNK = 2000
    nb = E // CHUNK
    row = edge_index[0].reshape(nb, 1, CHUNK)
    col = edge_index[1].reshape(nb, 1, CHUNK)

    aggr = pl.pallas_call(
        functools.partial(_scatter_body, chunk=CHUNK),
        grid=(nb,),
        in_specs=[
            pl.BlockSpec((1, 1, CHUNK), lambda i: (i, 0, 0), memory_space=pltpu.SMEM),
            pl.BlockSpec((1, 1, CHUNK), lambda i: (i, 0, 0), memory_space=pltpu.SMEM),
            pl.BlockSpec((N, C), lambda i: (0, 0)),
        ],
        out_specs=pl.BlockSpec((K, N, C), lambda i: (0, 0, 0)),
        out_shape=jax.ShapeDtypeStruct((K, N, C), jnp.float32),
        compiler_params=pltpu.CompilerParams(
            dimension_semantics=("arbitrary",)),
    )(row, col, x_flat)

    BN = 1000
    out = pl.pallas_call(
        _epilogue_body,
        grid=(N // BN,),
        in_specs=[
            pl.BlockSpec((K, BN, C), lambda i: (0, i, 0)),
            pl.BlockSpec((BN, C), lambda i: (i, 0)),
            pl.BlockSpec((C, C), lambda i: (0, 0)),
            pl.BlockSpec((1, C), lambda i: (0, 0)),
        ],
        out_specs=pl.BlockSpec((BN, C), lambda i: (i, 0)),
        out_shape=jax.ShapeDtypeStruct((N, C), jnp.float32),
    )(aggr, x_flat, W, b.reshape(1, C))
    return out.reshape(Bn, N, C)


# speedup vs baseline: 1.9650x; 1.8294x over previous
import functools

import jax
import jax.numpy as jnp
from jax.experimental import pallas as pl
from jax.experimental.pallas import tpu as pltpu

NEG_FILL = -1000000000.0
K = 4


def _scatter_body(row_ref, col_ref, x_ref, *aggr_refs, chunk):
    step = pl.program_id(0)

    @pl.when(step == 0)
    def _init():
        for k in range(K):
            aggr_refs[k][...] = jnp.full_like(aggr_refs[k][...], NEG_FILL)

    def body(i, carry):
        for k in range(K):
            r = row_ref[0, 0, i * K + k]
            c = col_ref[0, 0, i * K + k]
            xr = x_ref[c, :]
            aggr_refs[k][r, :] = jnp.maximum(aggr_refs[k][r, :], xr)
        return carry

    jax.lax.fori_loop(0, chunk // K, body, 0, unroll=2)


def _erf(z):
    s = jnp.sign(z)
    a = jnp.abs(z)
    t = 1.0 / (1.0 + 0.3275911 * a)
    poly = t * (0.254829592 + t * (-0.284496736 + t * (1.421413741
           + t * (-1.453152027 + t * 1.061405429))))
    return s * (1.0 - poly * jnp.exp(-a * a))


def _epilogue_body(*refs):
    t_refs = refs[:K]
    x_ref, w_ref, b_ref, out_ref = refs[K:]
    a = t_refs[0][...]
    for k in range(1, K):
        a = jnp.maximum(a, t_refs[k][...])
    a = jnp.where(a == NEG_FILL, 0.0, a) - x_ref[...]
    z = jnp.dot(a, w_ref[...], preferred_element_type=jnp.float32) + b_ref[...]
    out_ref[...] = 0.5 * z * (1.0 + _erf(z * 0.7071067811865476))


def kernel(x, edge_index, W, b):
    Bn, N, C = x.shape
    x_flat = x.reshape(N, C)
    E = edge_index.shape[1]
    CHUNK = 2000
    nb = E // CHUNK
    row = edge_index[0].reshape(nb, 1, CHUNK)
    col = edge_index[1].reshape(nb, 1, CHUNK)

    aggrs = pl.pallas_call(
        functools.partial(_scatter_body, chunk=CHUNK),
        grid=(nb,),
        in_specs=[
            pl.BlockSpec((1, 1, CHUNK), lambda i: (i, 0, 0), memory_space=pltpu.SMEM),
            pl.BlockSpec((1, 1, CHUNK), lambda i: (i, 0, 0), memory_space=pltpu.SMEM),
            pl.BlockSpec((N, C), lambda i: (0, 0)),
        ],
        out_specs=[pl.BlockSpec((N, C), lambda i: (0, 0)) for _ in range(K)],
        out_shape=[jax.ShapeDtypeStruct((N, C), jnp.float32) for _ in range(K)],
        compiler_params=pltpu.CompilerParams(
            dimension_semantics=("arbitrary",)),
    )(row, col, x_flat)

    BN = 1000
    out = pl.pallas_call(
        _epilogue_body,
        grid=(N // BN,),
        in_specs=(
            [pl.BlockSpec((BN, C), lambda i: (i, 0)) for _ in range(K)]
            + [
                pl.BlockSpec((BN, C), lambda i: (i, 0)),
                pl.BlockSpec((C, C), lambda i: (0, 0)),
                pl.BlockSpec((1, C), lambda i: (0, 0)),
            ]
        ),
        out_specs=pl.BlockSpec((BN, C), lambda i: (i, 0)),
        out_shape=jax.ShapeDtypeStruct((N, C), jnp.float32),
    )(*aggrs, x_flat, W, b.reshape(1, C))
    return out.reshape(Bn, N, C)


# speedup vs baseline: 2.7446x; 1.3968x over previous
import functools

import jax
import jax.numpy as jnp
from jax.experimental import pallas as pl
from jax.experimental.pallas import tpu as pltpu

NEG_FILL = -1000000000.0
K = 8


def _erf(z):
    s = jnp.sign(z)
    a = jnp.abs(z)
    t = 1.0 / (1.0 + 0.3275911 * a)
    poly = t * (0.254829592 + t * (-0.284496736 + t * (1.421413741
           + t * (-1.453152027 + t * 1.061405429))))
    return s * (1.0 - poly * jnp.exp(-a * a))


def _fused_body(row_ref, col_ref, x_ref, w_ref, b_ref, out_ref, *aggr_refs,
                chunk, nsteps):
    step = pl.program_id(0)

    @pl.when(step == 0)
    def _init():
        for k in range(K):
            aggr_refs[k][...] = jnp.full_like(aggr_refs[k][...], NEG_FILL)

    def body(i, carry):
        for k in range(K):
            r = row_ref[0, 0, i * K + k]
            c = col_ref[0, 0, i * K + k]
            xr = x_ref[c, :]
            aggr_refs[k][r, :] = jnp.maximum(aggr_refs[k][r, :], xr)
        return carry

    jax.lax.fori_loop(0, chunk // K, body, 0, unroll=8)

    @pl.when(step == nsteps - 1)
    def _epilogue():
        a = aggr_refs[0][...]
        for k in range(1, K):
            a = jnp.maximum(a, aggr_refs[k][...])
        a = jnp.where(a == NEG_FILL, 0.0, a) - x_ref[...]
        z = jnp.dot(a, w_ref[...],
                    preferred_element_type=jnp.float32) + b_ref[...]
        out_ref[...] = 0.5 * z * (1.0 + _erf(z * 0.7071067811865476))


def kernel(x, edge_index, W, b):
    Bn, N, C = x.shape
    x_flat = x.reshape(N, C)
    E = edge_index.shape[1]
    CHUNK = 2000
    nb = E // CHUNK
    row = edge_index[0].reshape(nb, 1, CHUNK)
    col = edge_index[1].reshape(nb, 1, CHUNK)

    out = pl.pallas_call(
        functools.partial(_fused_body, chunk=CHUNK, nsteps=nb),
        grid=(nb,),
        in_specs=[
            pl.BlockSpec((1, 1, CHUNK), lambda i: (i, 0, 0),
                         memory_space=pltpu.SMEM),
            pl.BlockSpec((1, 1, CHUNK), lambda i: (i, 0, 0),
                         memory_space=pltpu.SMEM),
            pl.BlockSpec((N, C), lambda i: (0, 0)),
            pl.BlockSpec((C, C), lambda i: (0, 0)),
            pl.BlockSpec((1, C), lambda i: (0, 0)),
        ],
        out_specs=pl.BlockSpec((N, C), lambda i: (0, 0)),
        out_shape=jax.ShapeDtypeStruct((N, C), jnp.float32),
        scratch_shapes=[pltpu.VMEM((N, C), jnp.float32) for _ in range(K)],
        compiler_params=pltpu.CompilerParams(
            dimension_semantics=("arbitrary",)),
    )(row, col, x_flat, W, b.reshape(1, C))
    return out.reshape(Bn, N, C)


# speedup vs baseline: 2.7632x; 1.0068x over previous
import functools

import jax
import jax.numpy as jnp
from jax.experimental import pallas as pl
from jax.experimental.pallas import tpu as pltpu

NEG_FILL = -1000000000.0
K = 8


def _erf(z):
    s = jnp.sign(z)
    a = jnp.abs(z)
    t = 1.0 / (1.0 + 0.3275911 * a)
    poly = t * (0.254829592 + t * (-0.284496736 + t * (1.421413741
           + t * (-1.453152027 + t * 1.061405429))))
    return s * (1.0 - poly * jnp.exp(-a * a))


def _fused_body(row_ref, col_ref, x_ref, w_ref, b_ref, out_ref, *aggr_refs,
                chunk, nsteps):
    step = pl.program_id(0)

    @pl.when(step == 0)
    def _init():
        for k in range(K):
            aggr_refs[k][...] = jnp.full_like(aggr_refs[k][...], NEG_FILL)

    def body(i, carry):
        for k in range(K):
            r = row_ref[0, 0, i * K + k]
            c = col_ref[0, 0, i * K + k]
            xr = x_ref[c, :]
            aggr_refs[k][r, :] = jnp.maximum(aggr_refs[k][r, :], xr)
        return carry

    jax.lax.fori_loop(0, chunk // K, body, 0, unroll=8)

    @pl.when(step == nsteps - 1)
    def _epilogue():
        a = aggr_refs[0][...]
        for k in range(1, K):
            a = jnp.maximum(a, aggr_refs[k][...])
        a = jnp.where(a == NEG_FILL, 0.0, a) - x_ref[...]
        z = jnp.dot(a, w_ref[...],
                    preferred_element_type=jnp.float32) + b_ref[...]
        out_ref[...] = 0.5 * z * (1.0 + _erf(z * 0.7071067811865476))


def kernel(x, edge_index, W, b):
    Bn, N, C = x.shape
    x_flat = x.reshape(N, C)
    E = edge_index.shape[1]
    CHUNK = 4000
    nb = E // CHUNK
    row = edge_index[0].reshape(nb, 1, CHUNK)
    col = edge_index[1].reshape(nb, 1, CHUNK)

    out = pl.pallas_call(
        functools.partial(_fused_body, chunk=CHUNK, nsteps=nb),
        grid=(nb,),
        in_specs=[
            pl.BlockSpec((1, 1, CHUNK), lambda i: (i, 0, 0),
                         memory_space=pltpu.SMEM),
            pl.BlockSpec((1, 1, CHUNK), lambda i: (i, 0, 0),
                         memory_space=pltpu.SMEM),
            pl.BlockSpec((N, C), lambda i: (0, 0)),
            pl.BlockSpec((C, C), lambda i: (0, 0)),
            pl.BlockSpec((1, C), lambda i: (0, 0)),
        ],
        out_specs=pl.BlockSpec((N, C), lambda i: (0, 0)),
        out_shape=jax.ShapeDtypeStruct((N, C), jnp.float32),
        scratch_shapes=[pltpu.VMEM((N, C), jnp.float32) for _ in range(K)],
        compiler_params=pltpu.CompilerParams(
            dimension_semantics=("arbitrary",)),
    )(row, col, x_flat, W, b.reshape(1, C))
    return out.reshape(Bn, N, C)


# speedup vs baseline: 2.8469x; 1.0303x over previous
import functools

import jax
import jax.numpy as jnp
from jax.experimental import pallas as pl
from jax.experimental.pallas import tpu as pltpu

NEG_FILL = -1000000000.0
K = 8


def _erf(z):
    s = jnp.sign(z)
    a = jnp.abs(z)
    t = 1.0 / (1.0 + 0.3275911 * a)
    poly = t * (0.254829592 + t * (-0.284496736 + t * (1.421413741
           + t * (-1.453152027 + t * 1.061405429))))
    return s * (1.0 - poly * jnp.exp(-a * a))


def _fused_body(row_ref, col_ref, x_ref, w_ref, b_ref, out_ref, *aggr_refs,
                chunk, nsteps):
    step = pl.program_id(0)

    @pl.when(step == 0)
    def _init():
        for k in range(K):
            aggr_refs[k][...] = jnp.full_like(aggr_refs[k][...], NEG_FILL)

    def body(i, carry):
        for k in range(K):
            r = row_ref[0, 0, i * K + k]
            c = col_ref[0, 0, i * K + k]
            xr = x_ref[c, :]
            aggr_refs[k][r, :] = jnp.maximum(aggr_refs[k][r, :], xr)
        return carry

    jax.lax.fori_loop(0, chunk // K, body, 0, unroll=16)

    @pl.when(step == nsteps - 1)
    def _epilogue():
        a = aggr_refs[0][...]
        for k in range(1, K):
            a = jnp.maximum(a, aggr_refs[k][...])
        a = jnp.where(a == NEG_FILL, 0.0, a) - x_ref[...]
        z = jnp.dot(a, w_ref[...],
                    preferred_element_type=jnp.float32) + b_ref[...]
        out_ref[...] = 0.5 * z * (1.0 + _erf(z * 0.7071067811865476))


def kernel(x, edge_index, W, b):
    Bn, N, C = x.shape
    x_flat = x.reshape(N, C)
    E = edge_index.shape[1]
    CHUNK = 4000
    nb = E // CHUNK
    row = edge_index[0].reshape(nb, 1, CHUNK)
    col = edge_index[1].reshape(nb, 1, CHUNK)

    out = pl.pallas_call(
        functools.partial(_fused_body, chunk=CHUNK, nsteps=nb),
        grid=(nb,),
        in_specs=[
            pl.BlockSpec((1, 1, CHUNK), lambda i: (i, 0, 0),
                         memory_space=pltpu.SMEM),
            pl.BlockSpec((1, 1, CHUNK), lambda i: (i, 0, 0),
                         memory_space=pltpu.SMEM),
            pl.BlockSpec((N, C), lambda i: (0, 0)),
            pl.BlockSpec((C, C), lambda i: (0, 0)),
            pl.BlockSpec((1, C), lambda i: (0, 0)),
        ],
        out_specs=pl.BlockSpec((N, C), lambda i: (0, 0)),
        out_shape=jax.ShapeDtypeStruct((N, C), jnp.float32),
        scratch_shapes=[pltpu.VMEM((N, C), jnp.float32) for _ in range(K)],
        compiler_params=pltpu.CompilerParams(
            dimension_semantics=("arbitrary",)),
    )(row, col, x_flat, W, b.reshape(1, C))
    return out.reshape(Bn, N, C)


# speedup vs baseline: 2.8771x; 1.0106x over previous
import functools

import jax
import jax.numpy as jnp
from jax.experimental import pallas as pl
from jax.experimental.pallas import tpu as pltpu

NEG_FILL = -1000000000.0
K = 8


def _erf(z):
    s = jnp.sign(z)
    a = jnp.abs(z)
    t = 1.0 / (1.0 + 0.3275911 * a)
    poly = t * (0.254829592 + t * (-0.284496736 + t * (1.421413741
           + t * (-1.453152027 + t * 1.061405429))))
    return s * (1.0 - poly * jnp.exp(-a * a))


def _fused_body(row_ref, col_ref, x_ref, w_ref, b_ref, out_ref, *aggr_refs,
                chunk, nsteps):
    step = pl.program_id(0)

    @pl.when(step == 0)
    def _init():
        for k in range(K):
            aggr_refs[k][...] = jnp.full_like(aggr_refs[k][...], NEG_FILL)

    def body(i, carry):
        for k in range(K):
            r = row_ref[0, 0, i * K + k]
            c = col_ref[0, 0, i * K + k]
            xr = x_ref[c, :]
            aggr_refs[k][r, :] = jnp.maximum(aggr_refs[k][r, :], xr)
        return carry

    jax.lax.fori_loop(0, chunk // K, body, 0, unroll=25)

    @pl.when(step == nsteps - 1)
    def _epilogue():
        a = aggr_refs[0][...]
        for k in range(1, K):
            a = jnp.maximum(a, aggr_refs[k][...])
        a = jnp.where(a == NEG_FILL, 0.0, a) - x_ref[...]
        z = jnp.dot(a, w_ref[...],
                    preferred_element_type=jnp.float32) + b_ref[...]
        out_ref[...] = 0.5 * z * (1.0 + _erf(z * 0.7071067811865476))


def kernel(x, edge_index, W, b):
    Bn, N, C = x.shape
    x_flat = x.reshape(N, C)
    E = edge_index.shape[1]
    CHUNK = 4000
    nb = E // CHUNK
    row = edge_index[0].reshape(nb, 1, CHUNK)
    col = edge_index[1].reshape(nb, 1, CHUNK)

    out = pl.pallas_call(
        functools.partial(_fused_body, chunk=CHUNK, nsteps=nb),
        grid=(nb,),
        in_specs=[
            pl.BlockSpec((1, 1, CHUNK), lambda i: (i, 0, 0),
                         memory_space=pltpu.SMEM),
            pl.BlockSpec((1, 1, CHUNK), lambda i: (i, 0, 0),
                         memory_space=pltpu.SMEM),
            pl.BlockSpec((N, C), lambda i: (0, 0)),
            pl.BlockSpec((C, C), lambda i: (0, 0)),
            pl.BlockSpec((1, C), lambda i: (0, 0)),
        ],
        out_specs=pl.BlockSpec((N, C), lambda i: (0, 0)),
        out_shape=jax.ShapeDtypeStruct((N, C), jnp.float32),
        scratch_shapes=[pltpu.VMEM((N, C), jnp.float32) for _ in range(K)],
        compiler_params=pltpu.CompilerParams(
            dimension_semantics=("arbitrary",)),
    )(row, col, x_flat, W, b.reshape(1, C))
    return out.reshape(Bn, N, C)
